# Initial kernel scaffold; baseline (speedup 1.0000x reference)
import jax
import jax.numpy as jnp
from jax.experimental import pallas as pl


def kernel(x, vec, edge_index, r_ij, f_ij, d_ij, batch, ln_w, ln_b, vec_ln_w, Wq, Wk, Wv, vec_proj_W, dv_proj_W, dv_proj_b, s_proj_W, s_proj_b, o_proj_W, o_proj_b):
    raise NotImplementedError("write your pallas kernel here")


# speedup vs baseline: 1.0174x; 1.0174x over previous
import functools

import jax
import jax.numpy as jnp
from jax import lax
from jax.experimental import pallas as pl

N = 10000
E = 160000
H = 128
NH = 8
HD = H // NH
B = 100
NPS = N // B
VDIM = 8
CUTOFF = 5.0

EB = 640
NB = 1000


def _attn_body(x_ref, lnw_ref, lnb_ref, wq_ref, wk_ref, wv_ref, v_ref):
    x = x_ref[0]
    m = jnp.mean(x, axis=-1, keepdims=True)
    var = jnp.mean((x - m) ** 2, axis=-1, keepdims=True)
    xn = (x - m) / jnp.sqrt(var + 1e-5) * lnw_ref[0] + lnb_ref[0]
    q = jnp.dot(xn, wq_ref[...], preferred_element_type=jnp.float32)
    k = jnp.dot(xn, wk_ref[...], preferred_element_type=jnp.float32)
    vv = jnp.dot(xn, wv_ref[...], preferred_element_type=jnp.float32)
    outs = []
    scale = 1.0 / (HD ** 0.5)
    for h in range(NH):
        sl = slice(h * HD, (h + 1) * HD)
        qh = q[:, sl]
        kh = k[:, sl]
        vh = vv[:, sl]
        sc = jnp.dot(qh, kh.T, preferred_element_type=jnp.float32) * scale
        sc = sc - jnp.max(sc, axis=-1, keepdims=True)
        e = jnp.exp(sc)
        a = e / jnp.sum(e, axis=-1, keepdims=True)
        outs.append(jnp.dot(a, vh, preferred_element_type=jnp.float32))
    v_ref[0] = jnp.concatenate(outs, axis=1) * (1.0 / NPS)


def _attention(x, ln_w, ln_b, Wq, Wk, Wv):
    xg = x.reshape(B, NPS, H)
    out = pl.pallas_call(
        _attn_body,
        grid=(B,),
        in_specs=[
            pl.BlockSpec((1, NPS, H), lambda g: (g, 0, 0)),
            pl.BlockSpec((1, H), lambda g: (0, 0)),
            pl.BlockSpec((1, H), lambda g: (0, 0)),
            pl.BlockSpec((H, H), lambda g: (0, 0)),
            pl.BlockSpec((H, H), lambda g: (0, 0)),
            pl.BlockSpec((H, H), lambda g: (0, 0)),
        ],
        out_specs=pl.BlockSpec((1, NPS, H), lambda g: (g, 0, 0)),
        out_shape=jax.ShapeDtypeStruct((B, NPS, H), jnp.float32),
    )(xg, ln_w.reshape(1, H), ln_b.reshape(1, H), Wq, Wk, Wv)
    return out.reshape(N, H)


def _silu_mm_body(a_ref, w_ref, b_ref, o_ref):
    y = jnp.dot(a_ref[...], w_ref[...], preferred_element_type=jnp.float32)
    y = y + b_ref[0]
    o_ref[...] = y * jax.nn.sigmoid(y)


def _silu_mm(a, w, b, blk):
    m, k = a.shape
    n = w.shape[1]
    return pl.pallas_call(
        _silu_mm_body,
        grid=(m // blk,),
        in_specs=[
            pl.BlockSpec((blk, k), lambda i: (i, 0)),
            pl.BlockSpec((k, n), lambda i: (0, 0)),
            pl.BlockSpec((1, n), lambda i: (0, 0)),
        ],
        out_specs=pl.BlockSpec((blk, n), lambda i: (i, 0)),
        out_shape=jax.ShapeDtypeStruct((m, n), jnp.float32),
    )(a, w, b.reshape(1, n))


def _mm_body(a_ref, w_ref, o_ref):
    o_ref[...] = jnp.dot(a_ref[...], w_ref[...],
                         preferred_element_type=jnp.float32)


def _mm(a, w, blk):
    m, k = a.shape
    n = w.shape[1]
    return pl.pallas_call(
        _mm_body,
        grid=(m // blk,),
        in_specs=[
            pl.BlockSpec((blk, k), lambda i: (i, 0)),
            pl.BlockSpec((k, n), lambda i: (0, 0)),
        ],
        out_specs=pl.BlockSpec((blk, n), lambda i: (i, 0)),
        out_shape=jax.ShapeDtypeStruct((m, n), jnp.float32),
    )(a, w)


def _vecsum_mm_body(vec_ref, w_ref, o_ref):
    s = jnp.sum(vec_ref[...], axis=1)
    o_ref[...] = jnp.dot(s, w_ref[...], preferred_element_type=jnp.float32)


def _vec_sum_mm(vec, w1):
    blk = NB
    return pl.pallas_call(
        _vecsum_mm_body,
        grid=(N // blk,),
        in_specs=[
            pl.BlockSpec((blk, VDIM, H), lambda i: (i, 0, 0)),
            pl.BlockSpec((H, H), lambda i: (0, 0)),
        ],
        out_specs=pl.BlockSpec((blk, H), lambda i: (i, 0)),
        out_shape=jax.ShapeDtypeStruct((N, H), jnp.float32),
    )(vec, w1)


def _edge_body(vsrc_ref, r_ref, dv_ref, w_ref, b_ref, lnw_ref,
               vj_ref, s1_ref, s2_ref):
    r = r_ref[...]
    cut = 0.5 * (jnp.cos(jnp.pi * r / CUTOFF) + 1.0) * (r < CUTOFF)
    vj = vsrc_ref[...] * cut * dv_ref[...]
    y = jnp.dot(vj, w_ref[...], preferred_element_type=jnp.float32) + b_ref[0]
    s = y * jax.nn.sigmoid(y)
    vj_ref[...] = vj
    s1_ref[...] = s[:, :H] * lnw_ref[0]
    s2_ref[...] = s[:, H:]


def _edge_stage(v_src, r_ij, dv, s_proj_W, s_proj_b, vec_ln_w):
    return pl.pallas_call(
        _edge_body,
        grid=(E // EB,),
        in_specs=[
            pl.BlockSpec((EB, H), lambda i: (i, 0)),
            pl.BlockSpec((EB, 1), lambda i: (i, 0)),
            pl.BlockSpec((EB, H), lambda i: (i, 0)),
            pl.BlockSpec((H, 2 * H), lambda i: (0, 0)),
            pl.BlockSpec((1, 2 * H), lambda i: (0, 0)),
            pl.BlockSpec((1, H), lambda i: (0, 0)),
        ],
        out_specs=[
            pl.BlockSpec((EB, H), lambda i: (i, 0)),
            pl.BlockSpec((EB, H), lambda i: (i, 0)),
            pl.BlockSpec((EB, H), lambda i: (i, 0)),
        ],
        out_shape=[
            jax.ShapeDtypeStruct((E, H), jnp.float32),
            jax.ShapeDtypeStruct((E, H), jnp.float32),
            jax.ShapeDtypeStruct((E, H), jnp.float32),
        ],
    )(v_src, r_ij.reshape(E, 1), dv, s_proj_W, s_proj_b.reshape(1, 2 * H),
      vec_ln_w.reshape(1, H))


def _final_body(xagg_ref, w_ref, b_ref, vsum_ref, vec3_ref, vagg_ref,
                dx_ref, dvec_ref):
    o = jnp.dot(xagg_ref[...], w_ref[...],
                preferred_element_type=jnp.float32) + b_ref[0]
    o1 = o[:, :H]
    o2 = o[:, H:2 * H]
    o3 = o[:, 2 * H:]
    dx_ref[...] = vsum_ref[...] * o2 + o3
    dvec_ref[...] = vec3_ref[...] * o1[:, None, :] + vagg_ref[...]


def _final(x_agg, o_proj_W, o_proj_b, vec_sum, vec3, vec_agg):
    return pl.pallas_call(
        _final_body,
        grid=(N // NB,),
        in_specs=[
            pl.BlockSpec((NB, H), lambda i: (i, 0)),
            pl.BlockSpec((H, 3 * H), lambda i: (0, 0)),
            pl.BlockSpec((1, 3 * H), lambda i: (0, 0)),
            pl.BlockSpec((NB, H), lambda i: (i, 0)),
            pl.BlockSpec((NB, VDIM, H), lambda i: (i, 0, 0)),
            pl.BlockSpec((NB, VDIM, H), lambda i: (i, 0, 0)),
        ],
        out_specs=[
            pl.BlockSpec((NB, H), lambda i: (i, 0)),
            pl.BlockSpec((NB, VDIM, H), lambda i: (i, 0, 0)),
        ],
        out_shape=[
            jax.ShapeDtypeStruct((N, H), jnp.float32),
            jax.ShapeDtypeStruct((N, VDIM, H), jnp.float32),
        ],
    )(x_agg, o_proj_W, o_proj_b.reshape(1, 3 * H), vec_sum, vec3, vec_agg)


def kernel(x, vec, edge_index, r_ij, f_ij, d_ij, batch, ln_w, ln_b, vec_ln_w,
           Wq, Wk, Wv, vec_proj_W, dv_proj_W, dv_proj_b,
           s_proj_W, s_proj_b, o_proj_W, o_proj_b):
    src = edge_index[0]
    dst = edge_index[1]

    v = _attention(x, ln_w, ln_b, Wq, Wk, Wv)
    dv = _silu_mm(f_ij, dv_proj_W, dv_proj_b, EB)
    w_scaled = vec_ln_w[:, None] * vec_proj_W
    vec_sum = _vec_sum_mm(vec, w_scaled[:, :H])
    vec3 = _mm(vec.reshape(N * VDIM, H), w_scaled[:, H:],
               EB).reshape(N, VDIM, H)

    v_src = jnp.take(v, src, axis=0)
    v_j, s1, s2 = _edge_stage(v_src, r_ij, dv, s_proj_W, s_proj_b, vec_ln_w)

    vec_src = jnp.take(vec, src, axis=0)
    vec_j = vec_src * s1[:, None, :] + s2[:, None, :] * d_ij[:, :, None]
    x_agg = jax.ops.segment_sum(v_j, dst, num_segments=N)
    vec_agg = jax.ops.segment_sum(vec_j, dst, num_segments=N)

    dx, dvec = _final(x_agg, o_proj_W, o_proj_b, vec_sum, vec3, vec_agg)
    return (dx, dvec)
